# baseline (speedup 1.0000x reference)
import functools

import jax
import jax.numpy as jnp
from jax import lax
from jax.experimental import pallas as pl
from jax.experimental.pallas import tpu as pltpu
from jax.experimental.pallas import tpu_sc as plsc

_B, _L, _S, _V = 16, 2048, 16384, 50000
_NC, _NS = 2, 16
_NW = _NC * _NS
_CHUNK = _S // _NW
_LANES = 16
_SC_UNROLL = 4
_LU_UNROLL = 4


def _membership(text_tokens, sub_tokens, sub_batch_ids):
    mesh = plsc.VectorSubcoreMesh(core_axis_name="c", subcore_axis_name="s")

    @functools.partial(
        pl.kernel,
        mesh=mesh,
        out_type=jax.ShapeDtypeStruct((_S,), jnp.bool_),
        compiler_params=pltpu.CompilerParams(needs_layout_passes=False),
        scratch_types=[
            pltpu.VMEM((_V,), jnp.int32),
            pltpu.VMEM((2, _L), jnp.int32),
            pltpu.VMEM((_CHUNK,), jnp.int32),
            pltpu.VMEM((_CHUNK,), jnp.int32),
            pltpu.VMEM((_CHUNK,), jnp.int32),
            pltpu.SemaphoreType.DMA,
            pltpu.SemaphoreType.DMA,
            pltpu.SemaphoreType.DMA,
        ],
    )
    def k(text_hbm, stok_hbm, sid_hbm, out_hbm,
          table, text_v, stok_v, sid_v, out_v, sem_tok, sem_sid, sem_t):
        wid = lax.axis_index("s") * _NC + lax.axis_index("c")
        base = wid * _CHUNK
        with jax.named_scope("ph_stage"):
            cp_sid = pltpu.async_copy(sid_hbm.at[pl.ds(base, _CHUNK)], sid_v, sem_sid)
            cp_tok = pltpu.async_copy(stok_hbm.at[pl.ds(base, _CHUNK)], stok_v, sem_tok)
            cp_sid.wait()

        with jax.named_scope("ph_rfirst"):
            r_first = sid_v[pl.ds(0, _LANES)][0]
            r_last = sid_v[pl.ds(_CHUNK - _LANES, _LANES)][_LANES - 1]
            pltpu.async_copy(text_hbm.at[r_first], text_v.at[0], sem_t)
            cp_tok.wait()

        zeros = jnp.zeros((_LANES,), jnp.int32)
        lane = lax.iota(jnp.int32, _LANES)

        def clear_probes(i, cc):
            for u in range(_LU_UNROLL):
                st = stok_v[pl.ds((i * _LU_UNROLL + u) * _LANES, _LANES)]
                plsc.store_scatter(table, [st], zeros)
            return cc

        with jax.named_scope("ph_clear"):
            lax.fori_loop(0, _CHUNK // (_LANES * _LU_UNROLL), clear_probes, 0)

        def row_body(r, c):
            sel = (r - r_first) & 1
            tag = r + 1
            tag_vec = zeros + tag
            with jax.named_scope("ph_textwait"):
                pltpu.make_async_copy(text_hbm.at[r], text_v.at[sel], sem_t).wait()

            @pl.when(r < r_last)
            def _prefetch():
                pltpu.async_copy(text_hbm.at[r + 1], text_v.at[1 - sel], sem_t)

            def scatter_tags(i, cc):
                for u in range(_SC_UNROLL):
                    toks = text_v[sel, pl.ds((i * _SC_UNROLL + u) * _LANES, _LANES)]
                    plsc.store_scatter(table, [toks], tag_vec)
                return cc

            with jax.named_scope("ph_scatter"):
                lax.fori_loop(0, _L // (_LANES * _SC_UNROLL), scatter_tags, 0)

            def lookup(i, cc):
                for u in range(_LU_UNROLL):
                    off = (i * _LU_UNROLL + u) * _LANES
                    st = stok_v[pl.ds(off, _LANES)]
                    si = sid_v[pl.ds(off, _LANES)]
                    g = plsc.load_gather(table, [st])
                    found = (g == tag).astype(jnp.int32)
                    plsc.store_scatter(out_v, [lane + off], found, mask=si == r)
                return cc

            with jax.named_scope("ph_lookup"):
                lax.fori_loop(0, _CHUNK // (_LANES * _LU_UNROLL), lookup, 0)
            return c

        lax.fori_loop(r_first, r_last + 1, row_body, 0)

        with jax.named_scope("ph_out"):
            pltpu.sync_copy(out_v, out_hbm.at[pl.ds(base, _CHUNK)])

    return k(text_tokens, sub_tokens, sub_batch_ids)


def kernel(text_tokens, sub_tokens, sub_batch_ids):
    return _membership(text_tokens, sub_tokens, sub_batch_ids)


# speedup vs baseline: 1.0600x; 1.0600x over previous
import functools

import jax
import jax.numpy as jnp
from jax import lax
from jax.experimental import pallas as pl
from jax.experimental.pallas import tpu as pltpu
from jax.experimental.pallas import tpu_sc as plsc

_B, _L, _S, _V = 16, 2048, 16384, 50000
_NC, _NS = 2, 16
_NW = _NC * _NS
_CHUNK = _S // _NW
_LANES = 16


def _membership(text_tokens, sub_tokens, sub_batch_ids):
    mesh = plsc.VectorSubcoreMesh(core_axis_name="c", subcore_axis_name="s")

    @functools.partial(
        pl.kernel,
        mesh=mesh,
        out_type=jax.ShapeDtypeStruct((_S,), jnp.bool_),
        compiler_params=pltpu.CompilerParams(needs_layout_passes=False),
        scratch_types=[
            pltpu.VMEM((_V,), jnp.int32),
            pltpu.VMEM((2, _L), jnp.int32),
            pltpu.VMEM((_CHUNK,), jnp.int32),
            pltpu.VMEM((_CHUNK,), jnp.int32),
            pltpu.VMEM((_CHUNK,), jnp.int32),
            pltpu.SemaphoreType.DMA,
            pltpu.SemaphoreType.DMA,
            pltpu.SemaphoreType.DMA,
        ],
    )
    def k(text_hbm, stok_hbm, sid_hbm, out_hbm,
          table, text_v, stok_v, sid_v, out_v, sem_tok, sem_sid, sem_t):
        wid = lax.axis_index("s") * _NC + lax.axis_index("c")
        base = wid * _CHUNK
        cp_sid = pltpu.async_copy(sid_hbm.at[pl.ds(base, _CHUNK)], sid_v, sem_sid)
        cp_tok = pltpu.async_copy(stok_hbm.at[pl.ds(base, _CHUNK)], stok_v, sem_tok)
        cp_sid.wait()

        r_first = sid_v[pl.ds(0, _LANES)][0]
        r_last = sid_v[pl.ds(_CHUNK - _LANES, _LANES)][_LANES - 1]
        pltpu.async_copy(text_hbm.at[r_first], text_v.at[0], sem_t)
        cp_tok.wait()

        zeros = jnp.zeros((_LANES,), jnp.int32)
        lane = lax.iota(jnp.int32, _LANES)

        @plsc.parallel_loop(0, _CHUNK // _LANES, unroll=8)
        def _clear(i):
            st = stok_v[pl.ds(i * _LANES, _LANES)]
            plsc.store_scatter(table, [st], zeros)

        def row_body(r, c):
            sel = (r - r_first) & 1
            tag = r + 1
            tag_vec = zeros + tag
            pltpu.make_async_copy(text_hbm.at[r], text_v.at[sel], sem_t).wait()

            @pl.when(r < r_last)
            def _prefetch():
                pltpu.async_copy(text_hbm.at[r + 1], text_v.at[1 - sel], sem_t)

            @plsc.parallel_loop(0, _L // _LANES, unroll=8)
            def _scatter(i):
                toks = text_v[sel, pl.ds(i * _LANES, _LANES)]
                plsc.store_scatter(table, [toks], tag_vec)

            @plsc.parallel_loop(0, _CHUNK // _LANES, unroll=8)
            def _lookup(i):
                off = i * _LANES
                st = stok_v[pl.ds(off, _LANES)]
                si = sid_v[pl.ds(off, _LANES)]
                g = plsc.load_gather(table, [st])
                found = (g == tag).astype(jnp.int32)
                plsc.store_scatter(out_v, [lane + off], found, mask=si == r)

            return c

        lax.fori_loop(r_first, r_last + 1, row_body, 0)

        pltpu.sync_copy(out_v, out_hbm.at[pl.ds(base, _CHUNK)])

    return k(text_tokens, sub_tokens, sub_batch_ids)


def kernel(text_tokens, sub_tokens, sub_batch_ids):
    return _membership(text_tokens, sub_tokens, sub_batch_ids)
